# baseline (speedup 1.0000x reference)
import functools

import jax
import jax.numpy as jnp
from jax import lax
from jax.experimental import pallas as pl
from jax.experimental.pallas import tpu as pltpu
from jax.experimental.pallas import tpu_sc as plsc

_D = 64
_NC = 2
_NS = 16
_NW = _NC * _NS
_R = 512


def _sc_gather(table, idx):
    n = idx.shape[0]
    bpw = n // _NW
    nchunk = bpw // _R
    mesh = plsc.VectorSubcoreMesh(core_axis_name="c", subcore_axis_name="s")

    @functools.partial(
        pl.kernel,
        out_type=jax.ShapeDtypeStruct((n, _D), jnp.float32),
        mesh=mesh,
        scratch_types=[
            pltpu.VMEM((bpw,), jnp.int32),
            pltpu.VMEM((_R, _D), jnp.float32),
            pltpu.SemaphoreType.DMA,
        ],
        compiler_params=pltpu.CompilerParams(use_tc_tiling_on_sc=False),
    )
    def body(table_hbm, idx_hbm, out_hbm, idx_v, rows_v, gsem):
        wid = lax.axis_index("s") * _NC + lax.axis_index("c")
        base = pl.multiple_of(wid * bpw, 8)
        pltpu.sync_copy(idx_hbm.at[pl.ds(base, bpw)], idx_v)

        @pl.loop(0, nchunk)
        def chunk(c):
            off = pl.multiple_of(c * _R, 8)
            pltpu.async_copy(
                table_hbm.at[idx_v.at[pl.ds(off, _R)]], rows_v, gsem
            ).wait()
            pltpu.sync_copy(rows_v, out_hbm.at[pl.ds(base + off, _R)])

    return body(table, idx)


def kernel(feats, ids):
    b, k = ids.shape
    flat = ids.reshape(-1).astype(jnp.int32)
    out = _sc_gather(feats, flat)
    return out.reshape(b, k, _D)


# speedup vs baseline: 1.0138x; 1.0138x over previous
import functools

import jax
import jax.numpy as jnp
from jax import lax
from jax.experimental import pallas as pl
from jax.experimental.pallas import tpu as pltpu
from jax.experimental.pallas import tpu_sc as plsc

_D = 64
_NC = 2
_NS = 16
_NW = _NC * _NS
_R = 512


def _sc_gather(table, idx):
    n = idx.shape[0]
    bpw = n // _NW
    nchunk = bpw // _R
    assert nchunk % 2 == 0 and nchunk >= 4
    mesh = plsc.VectorSubcoreMesh(core_axis_name="c", subcore_axis_name="s")

    @functools.partial(
        pl.kernel,
        out_type=jax.ShapeDtypeStruct((n, _D), jnp.float32),
        mesh=mesh,
        scratch_types=[
            pltpu.VMEM((bpw,), jnp.int32),
            pltpu.VMEM((_R, _D), jnp.float32),
            pltpu.VMEM((_R, _D), jnp.float32),
            pltpu.SemaphoreType.DMA,
            pltpu.SemaphoreType.DMA,
        ],
        compiler_params=pltpu.CompilerParams(use_tc_tiling_on_sc=False),
    )
    def body(table_hbm, idx_hbm, out_hbm, idx_v, rows0, rows1, gsem0, gsem1):
        wid = lax.axis_index("s") * _NC + lax.axis_index("c")
        base = pl.multiple_of(wid * bpw, 8)
        pltpu.sync_copy(idx_hbm.at[pl.ds(base, bpw)], idx_v)

        def idx_slice(c):
            return idx_v.at[pl.ds(pl.multiple_of(c * _R, 8), _R)]

        def start_gather(c, buf, sem):
            pltpu.async_copy(table_hbm.at[idx_slice(c)], buf, sem)

        def wait_gather(c, buf, sem):
            pltpu.make_async_copy(table_hbm.at[idx_slice(c)], buf, sem).wait()

        def copy_out(c, buf):
            off = pl.multiple_of(base + c * _R, 8)
            pltpu.sync_copy(buf, out_hbm.at[pl.ds(off, _R)])

        start_gather(0, rows0, gsem0)
        start_gather(1, rows1, gsem1)
        wait_gather(0, rows0, gsem0)
        copy_out(0, rows0)

        @pl.loop(1, nchunk - 1, step=2)
        def pair(c):
            start_gather(c + 1, rows0, gsem0)
            wait_gather(c, rows1, gsem1)
            copy_out(c, rows1)
            start_gather(c + 2, rows1, gsem1)
            wait_gather(c + 1, rows0, gsem0)
            copy_out(c + 1, rows0)

        wait_gather(nchunk - 1, rows1, gsem1)
        copy_out(nchunk - 1, rows1)

    return body(table, idx)


def kernel(feats, ids):
    b, k = ids.shape
    flat = ids.reshape(-1).astype(jnp.int32)
    out = _sc_gather(feats, flat)
    return out.reshape(b, k, _D)


# speedup vs baseline: 1.0390x; 1.0248x over previous
import functools

import jax
import jax.numpy as jnp
from jax import lax
from jax.experimental import pallas as pl
from jax.experimental.pallas import tpu as pltpu
from jax.experimental.pallas import tpu_sc as plsc

_D = 64
_NC = 2
_NS = 16
_NW = _NC * _NS
_R = 256
_W = 128


def _sc_gather(table, idx):
    n = idx.shape[0]
    bpw = n // _NW
    nchunk = bpw // _R
    assert nchunk % 2 == 0 and nchunk >= 4
    mesh = plsc.VectorSubcoreMesh(core_axis_name="c", subcore_axis_name="s")

    @functools.partial(
        pl.kernel,
        out_type=jax.ShapeDtypeStruct((n, _D), jnp.float32),
        mesh=mesh,
        scratch_types=[
            pltpu.VMEM((bpw,), jnp.int32),
            pltpu.VMEM((_R, _W), jnp.float32),
            pltpu.VMEM((_R, _W), jnp.float32),
            pltpu.SemaphoreType.DMA,
            pltpu.SemaphoreType.DMA,
        ],
        compiler_params=pltpu.CompilerParams(use_tc_tiling_on_sc=False),
    )
    def body(table_hbm, idx_hbm, out_hbm, idx_v, rows0, rows1, gsem0, gsem1):
        wid = lax.axis_index("s") * _NC + lax.axis_index("c")
        base = pl.multiple_of(wid * bpw, 8)
        pltpu.sync_copy(idx_hbm.at[pl.ds(base, bpw)], idx_v)

        def idx_slice(c):
            return idx_v.at[pl.ds(pl.multiple_of(c * _R, 8), _R)]

        def start_gather(c, buf, sem):
            pltpu.async_copy(table_hbm.at[idx_slice(c)], buf, sem)

        def wait_gather(c, buf, sem):
            pltpu.make_async_copy(table_hbm.at[idx_slice(c)], buf, sem).wait()

        def copy_out(c, buf):
            off = pl.multiple_of(base + c * _R, 8)
            pltpu.sync_copy(buf.at[:, pl.ds(0, _D)], out_hbm.at[pl.ds(off, _R)])

        start_gather(0, rows0, gsem0)
        start_gather(1, rows1, gsem1)
        wait_gather(0, rows0, gsem0)
        copy_out(0, rows0)

        @pl.loop(1, nchunk - 1, step=2)
        def pair(c):
            start_gather(c + 1, rows0, gsem0)
            wait_gather(c, rows1, gsem1)
            copy_out(c, rows1)
            start_gather(c + 2, rows1, gsem1)
            wait_gather(c + 1, rows0, gsem0)
            copy_out(c + 1, rows0)

        wait_gather(nchunk - 1, rows1, gsem1)
        copy_out(nchunk - 1, rows1)

    return body(table, idx)


def kernel(feats, ids):
    b, k = ids.shape
    flat = ids.reshape(-1).astype(jnp.int32)
    feats_pad = jnp.pad(feats, ((0, 0), (0, _W - _D)))
    out = _sc_gather(feats_pad, flat)
    return out.reshape(b, k, _D)
